# Initial kernel scaffold; baseline (speedup 1.0000x reference)
import jax
import jax.numpy as jnp
from jax.experimental import pallas as pl


def kernel(x, edge_index0, edge_index1, W_fc, b_fc, W_self0, W_neigh0, b0, W_self1, W_neigh1, b1):
    raise NotImplementedError("write your pallas kernel here")


# speedup vs baseline: 7.7563x; 7.7563x over previous
import functools

import jax
import jax.numpy as jnp
from jax import lax
from jax.experimental import pallas as pl
from jax.experimental.pallas import tpu as pltpu
from jax.experimental.pallas import tpu_sc as plsc

_N = 10000
_E = 320000
_D = 128
_H = 128
_C = 16
_NPAD = 10240
_NC = 2
_NS = 16
_NW = _NC * _NS
_K = 128
_RPT = _NPAD // _NS

_CH0 = 157
_CH1 = 79


def _zero_fill(ref, ncol):
    zv = jnp.zeros((16,), jnp.float32)

    def zrow(r, _):
        for j in range(ncol // 16):
            ref[r, pl.ds(j * 16, 16)] = zv
        return 0

    lax.fori_loop(0, _K, zrow, 0)


def _one_fill(ref):
    ov = jnp.ones((16,), jnp.float32)

    def orow(r, _):
        ref[r, :] = ov
        return 0

    lax.fori_loop(0, _K, orow, 0)


@functools.partial(
    pl.kernel,
    out_type=(jax.ShapeDtypeStruct((_NC, _NPAD, 64), jnp.float32),
              jax.ShapeDtypeStruct((_NPAD, 16), jnp.float32)),
    mesh=plsc.VectorSubcoreMesh(core_axis_name="c", subcore_axis_name="s"),
    compiler_params=pltpu.CompilerParams(use_tc_tiling_on_sc=False),
    scratch_types=[
        pltpu.VMEM((_CH0, _K), jnp.int32),
        pltpu.VMEM((_CH0, _K), jnp.int32),
        pltpu.VMEM((_K, 64), jnp.float32),
        pltpu.VMEM((_K, 16), jnp.float32),
        pltpu.VMEM_SHARED((_NPAD, 64), jnp.float32),
        pltpu.VMEM_SHARED((_NPAD, 16), jnp.float32),
        pltpu.SemaphoreType.DMA,
    ],
)
def _agg0(h_hbm, src_hbm, dst_hbm, acc_out, deg_out,
          src_v, dst_v, buf, ones_v, acc_sh, deg_sh, gsem):
    cid = lax.axis_index("c")
    sid = lax.axis_index("s")

    pltpu.sync_copy(src_hbm.at[sid], src_v)
    pltpu.sync_copy(dst_hbm.at[sid], dst_v)

    _zero_fill(buf, 64)
    _zero_fill(ones_v, 16)
    base = sid * _RPT
    for t in range(_RPT // _K):
        pltpu.sync_copy(buf, acc_sh.at[pl.ds(base + t * _K, _K), pl.ds(0, 64)])
        pltpu.sync_copy(ones_v, deg_sh.at[pl.ds(base + t * _K, _K)])
    _one_fill(ones_v)

    plsc.subcore_barrier()

    hview = h_hbm.at[cid]
    on_core0 = cid == 0

    def body(j, _):
        pltpu.async_copy(hview.at[src_v.at[j]], buf, gsem).wait()
        pltpu.sync_copy(buf, acc_sh.at[dst_v.at[j]], add=True)

        @pl.when(on_core0)
        def _():
            pltpu.sync_copy(ones_v, deg_sh.at[dst_v.at[j]], add=True)

        return 0

    lax.fori_loop(0, _CH0, body, 0)

    plsc.subcore_barrier()

    for t in range(_RPT // _K):
        pltpu.sync_copy(acc_sh.at[pl.ds(base + t * _K, _K)],
                        acc_out.at[cid, pl.ds(base + t * _K, _K)])

        @pl.when(on_core0)
        def _():
            pltpu.sync_copy(deg_sh.at[pl.ds(base + t * _K, _K)],
                            deg_out.at[pl.ds(base + t * _K, _K)])


@functools.partial(
    pl.kernel,
    out_type=(jax.ShapeDtypeStruct((_NC, _NPAD, 16), jnp.float32),
              jax.ShapeDtypeStruct((_NC, _NPAD, 16), jnp.float32)),
    mesh=plsc.VectorSubcoreMesh(core_axis_name="c", subcore_axis_name="s"),
    compiler_params=pltpu.CompilerParams(use_tc_tiling_on_sc=False),
    scratch_types=[
        pltpu.VMEM((_CH1, _K), jnp.int32),
        pltpu.VMEM((_CH1, _K), jnp.int32),
        pltpu.VMEM((_K, 16), jnp.float32),
        pltpu.VMEM((_K, 16), jnp.float32),
        pltpu.VMEM_SHARED((_NPAD, 16), jnp.float32),
        pltpu.VMEM_SHARED((_NPAD, 16), jnp.float32),
        pltpu.SemaphoreType.DMA,
    ],
)
def _agg1(g_hbm, src_hbm, dst_hbm, acc_out, deg_out,
          src_v, dst_v, buf, ones_v, acc_sh, deg_sh, gsem):
    cid = lax.axis_index("c")
    sid = lax.axis_index("s")
    wid = sid * _NC + cid

    pltpu.sync_copy(src_hbm.at[wid], src_v)
    pltpu.sync_copy(dst_hbm.at[wid], dst_v)

    _zero_fill(buf, 16)
    _zero_fill(ones_v, 16)
    base = sid * _RPT
    for t in range(_RPT // _K):
        pltpu.sync_copy(buf, acc_sh.at[pl.ds(base + t * _K, _K)])
        pltpu.sync_copy(ones_v, deg_sh.at[pl.ds(base + t * _K, _K)])
    _one_fill(ones_v)

    plsc.subcore_barrier()

    def body(j, _):
        pltpu.async_copy(g_hbm.at[src_v.at[j]], buf, gsem).wait()
        pltpu.sync_copy(buf, acc_sh.at[dst_v.at[j]], add=True)
        pltpu.sync_copy(ones_v, deg_sh.at[dst_v.at[j]], add=True)
        return 0

    lax.fori_loop(0, _CH1, body, 0)

    plsc.subcore_barrier()

    for t in range(_RPT // _K):
        pltpu.sync_copy(acc_sh.at[pl.ds(base + t * _K, _K)],
                        acc_out.at[cid, pl.ds(base + t * _K, _K)])
        pltpu.sync_copy(deg_sh.at[pl.ds(base + t * _K, _K)],
                        deg_out.at[cid, pl.ds(base + t * _K, _K)])


def _prep_edges(ei, nsplit, chunks):
    tot = nsplit * chunks * _K
    fill = jnp.full((tot - _E,), _N, jnp.int32)
    src = jnp.concatenate([ei[0], fill]).reshape(nsplit, chunks, _K)
    dst = jnp.concatenate([ei[1], fill]).reshape(nsplit, chunks, _K)
    return src, dst


def _tc_fc(x, w, b):
    def body(x_ref, w_ref, b_ref, o_ref):
        r = jnp.maximum(
            jnp.dot(x_ref[...], w_ref[...],
                    preferred_element_type=jnp.float32) + b_ref[...], 0.0)
        o_ref[...] = jnp.stack([r[:, :64], r[:, 64:]], axis=0)

    return pl.pallas_call(
        body,
        grid=(10,),
        in_specs=[pl.BlockSpec((1024, _D), lambda i: (i, 0)),
                  pl.BlockSpec((_D, _H), lambda i: (0, 0)),
                  pl.BlockSpec((1, _H), lambda i: (0, 0))],
        out_specs=pl.BlockSpec((_NC, 1024, 64), lambda i: (0, i, 0)),
        out_shape=jax.ShapeDtypeStruct((_NC, _NPAD, 64), jnp.float32),
    )(x, w, b)


def _tc_mid(hs, acc, deg, ws0, wn0, b0, ws1, wn1):
    def body(h_ref, a_ref, d_ref, ws0_r, wn0_r, b0_r, ws1_r, wn1_r,
             s_ref, g_ref):
        h = jnp.concatenate([h_ref[0], h_ref[1]], axis=-1)
        d = jnp.maximum(d_ref[:, 0:1], 1.0)
        mean = jnp.concatenate([a_ref[0], a_ref[1]], axis=-1) / d
        h1 = jnp.maximum(
            jnp.dot(h, ws0_r[...], preferred_element_type=jnp.float32)
            + jnp.dot(mean, wn0_r[...], preferred_element_type=jnp.float32)
            + b0_r[...], 0.0)
        s_ref[...] = jnp.dot(h1, ws1_r[...], preferred_element_type=jnp.float32)
        g_ref[...] = jnp.dot(h1, wn1_r[...], preferred_element_type=jnp.float32)

    return pl.pallas_call(
        body,
        grid=(10,),
        in_specs=[pl.BlockSpec((_NC, 1024, 64), lambda i: (0, i, 0)),
                  pl.BlockSpec((_NC, 1024, 64), lambda i: (0, i, 0)),
                  pl.BlockSpec((1024, 16), lambda i: (i, 0)),
                  pl.BlockSpec((_H, _H), lambda i: (0, 0)),
                  pl.BlockSpec((_H, _H), lambda i: (0, 0)),
                  pl.BlockSpec((1, _H), lambda i: (0, 0)),
                  pl.BlockSpec((_H, _C), lambda i: (0, 0)),
                  pl.BlockSpec((_H, _C), lambda i: (0, 0))],
        out_specs=[pl.BlockSpec((1024, _C), lambda i: (i, 0)),
                   pl.BlockSpec((1024, _C), lambda i: (i, 0))],
        out_shape=[jax.ShapeDtypeStruct((_NPAD, _C), jnp.float32),
                   jax.ShapeDtypeStruct((_NPAD, _C), jnp.float32)],
    )(hs, acc, deg, ws0, wn0, b0, ws1, wn1)


def _tc_out(s2d, a2d, d2d, b1t):
    R = _NPAD // 8

    def body(s_ref, a_ref, d_ref, b_ref, o_ref):
        agg = (a_ref[0] + a_ref[1]) / jnp.maximum(d_ref[0] + d_ref[1], 1.0)
        o_ref[...] = s_ref[...] + agg + b_ref[...]

    return pl.pallas_call(
        body,
        grid=(1,),
        in_specs=[pl.BlockSpec((R, 128), lambda i: (0, 0)),
                  pl.BlockSpec((_NC, R, 128), lambda i: (0, 0, 0)),
                  pl.BlockSpec((_NC, R, 128), lambda i: (0, 0, 0)),
                  pl.BlockSpec((1, 128), lambda i: (0, 0))],
        out_specs=pl.BlockSpec((R, 128), lambda i: (0, 0)),
        out_shape=jax.ShapeDtypeStruct((R, 128), jnp.float32),
    )(s2d, a2d, d2d, b1t)


def kernel(x, edge_index0, edge_index1, W_fc, b_fc,
           W_self0, W_neigh0, b0, W_self1, W_neigh1, b1):
    xp = jnp.zeros((_NPAD, _D), jnp.float32).at[:_N].set(x)
    src0, dst0 = _prep_edges(edge_index0, _NS, _CH0)
    src1, dst1 = _prep_edges(edge_index1, _NW, _CH1)

    hs = _tc_fc(xp, W_fc, b_fc.reshape(1, _H))
    acc0, deg0 = _agg0(hs, src0, dst0)
    s, g = _tc_mid(hs, acc0, deg0, W_self0, W_neigh0, b0.reshape(1, _H),
                   W_self1, W_neigh1)
    acc1, deg1 = _agg1(g, src1, dst1)
    out2d = _tc_out(s.reshape(_NPAD // 8, 128),
                    acc1.reshape(_NC, _NPAD // 8, 128),
                    deg1.reshape(_NC, _NPAD // 8, 128),
                    jnp.tile(b1, 8).reshape(1, 128))
    return out2d.reshape(_NPAD, _C)[:_N]


# speedup vs baseline: 7.8998x; 1.0185x over previous
import functools

import jax
import jax.numpy as jnp
from jax import lax
from jax.experimental import pallas as pl
from jax.experimental.pallas import tpu as pltpu
from jax.experimental.pallas import tpu_sc as plsc

_N = 10000
_E = 320000
_D = 128
_H = 128
_C = 16
_NPAD = 10240
_NC = 2
_NS = 16
_NW = _NC * _NS
_K = 128
_RPT = _NPAD // _NS
_NBUF = 4

_CH0 = 160
_CH1 = 80


def _zero_fill(ref, ncol):
    zv = jnp.zeros((16,), jnp.float32)

    def zrow(r, _):
        for j in range(ncol // 16):
            ref[r, pl.ds(j * 16, 16)] = zv
        return 0

    lax.fori_loop(0, _K, zrow, 0)


def _one_fill(ref):
    ov = jnp.ones((16,), jnp.float32)

    def orow(r, _):
        ref[r, :] = ov
        return 0

    lax.fori_loop(0, _K, orow, 0)


@functools.partial(
    pl.kernel,
    out_type=(jax.ShapeDtypeStruct((_NC, _NPAD, 64), jnp.float32),
              jax.ShapeDtypeStruct((_NC, _NPAD, 16), jnp.float32)),
    mesh=plsc.VectorSubcoreMesh(core_axis_name="c", subcore_axis_name="s"),
    compiler_params=pltpu.CompilerParams(use_tc_tiling_on_sc=False),
    scratch_types=[
        pltpu.VMEM((_CH0, _K), jnp.int32),
        pltpu.VMEM((_CH0, _K), jnp.int32),
        pltpu.VMEM((_NBUF, _K, 64), jnp.float32),
        pltpu.VMEM((_K, 16), jnp.float32),
        pltpu.VMEM_SHARED((_NPAD, 64), jnp.float32),
        pltpu.VMEM_SHARED((_NPAD, 16), jnp.float32),
        pltpu.SemaphoreType.DMA,
        pltpu.SemaphoreType.DMA,
        pltpu.SemaphoreType.DMA,
    ],
)
def _agg0(h_hbm, src_hbm, dst_hbm, acc_out, deg_out,
          src_v, dst_v, buf, ones_v, acc_sh, deg_sh,
          gsem, ssem, dsem):
    cid = lax.axis_index("c")
    sid = lax.axis_index("s")
    on_core0 = cid == 0
    on_core1 = cid == 1

    pltpu.sync_copy(src_hbm.at[sid], src_v)
    pltpu.sync_copy(dst_hbm.at[sid], dst_v)

    zbuf = buf.at[0]
    _zero_fill(zbuf, 64)
    _zero_fill(ones_v, 16)
    base = sid * _RPT
    for t in range(_RPT // _K):
        pltpu.sync_copy(zbuf, acc_sh.at[pl.ds(base + t * _K, _K), pl.ds(0, 64)])
        pltpu.sync_copy(ones_v, deg_sh.at[pl.ds(base + t * _K, _K)])
    _one_fill(ones_v)

    hview = h_hbm.at[cid]

    for b in range(_NBUF):
        pltpu.async_copy(hview.at[src_v.at[b]], buf.at[b], gsem)
    plsc.subcore_barrier()

    def group(g, _):
        for b in range(_NBUF):
            j = g * _NBUF + b
            pltpu.make_async_copy(hview.at[src_v.at[0]], buf.at[b],
                                  gsem).wait()
            pltpu.async_copy(buf.at[b], acc_sh.at[dst_v.at[j]], ssem,
                             add=True)

            @pl.when((on_core0 & (j < _CH0 // 2))
                     | (on_core1 & (j >= _CH0 // 2)))
            def _():
                pltpu.async_copy(ones_v, deg_sh.at[dst_v.at[j]], dsem,
                                 add=True)

            pltpu.make_async_copy(buf.at[b], acc_sh.at[dst_v.at[0]],
                                  ssem).wait()

            @pl.when(g + 1 < _CH0 // _NBUF)
            def _():
                pltpu.async_copy(hview.at[src_v.at[j + _NBUF]], buf.at[b],
                                 gsem)
        return 0

    lax.fori_loop(0, _CH0 // _NBUF, group, 0)

    def ddrain(j, _):
        pltpu.make_async_copy(ones_v, deg_sh.at[dst_v.at[0]], dsem).wait()
        return 0

    lax.fori_loop(0, _CH0 // 2, ddrain, 0)

    plsc.subcore_barrier()

    for t in range(_RPT // _K):
        pltpu.sync_copy(acc_sh.at[pl.ds(base + t * _K, _K)],
                        acc_out.at[cid, pl.ds(base + t * _K, _K)])

        pltpu.sync_copy(deg_sh.at[pl.ds(base + t * _K, _K)],
                        deg_out.at[cid, pl.ds(base + t * _K, _K)])


@functools.partial(
    pl.kernel,
    out_type=(jax.ShapeDtypeStruct((_NC, _NPAD, 16), jnp.float32),
              jax.ShapeDtypeStruct((_NC, _NPAD, 16), jnp.float32)),
    mesh=plsc.VectorSubcoreMesh(core_axis_name="c", subcore_axis_name="s"),
    compiler_params=pltpu.CompilerParams(use_tc_tiling_on_sc=False),
    scratch_types=[
        pltpu.VMEM((_CH1, _K), jnp.int32),
        pltpu.VMEM((_CH1, _K), jnp.int32),
        pltpu.VMEM((_NBUF, _K, 16), jnp.float32),
        pltpu.VMEM((_K, 16), jnp.float32),
        pltpu.VMEM_SHARED((_NPAD, 16), jnp.float32),
        pltpu.VMEM_SHARED((_NPAD, 16), jnp.float32),
        pltpu.SemaphoreType.DMA,
        pltpu.SemaphoreType.DMA,
        pltpu.SemaphoreType.DMA,
    ],
)
def _agg1(g_hbm, src_hbm, dst_hbm, acc_out, deg_out,
          src_v, dst_v, buf, ones_v, acc_sh, deg_sh, gsem, ssem, dsem):
    cid = lax.axis_index("c")
    sid = lax.axis_index("s")
    wid = sid * _NC + cid

    pltpu.sync_copy(src_hbm.at[wid], src_v)
    pltpu.sync_copy(dst_hbm.at[wid], dst_v)

    zbuf = buf.at[0]
    _zero_fill(zbuf, 16)
    _zero_fill(ones_v, 16)
    base = sid * _RPT
    for t in range(_RPT // _K):
        pltpu.sync_copy(zbuf, acc_sh.at[pl.ds(base + t * _K, _K)])
        pltpu.sync_copy(ones_v, deg_sh.at[pl.ds(base + t * _K, _K)])
    _one_fill(ones_v)

    for b in range(_NBUF):
        pltpu.async_copy(g_hbm.at[src_v.at[b]], buf.at[b], gsem)
    plsc.subcore_barrier()

    def group(g, _):
        for b in range(_NBUF):
            j = g * _NBUF + b
            pltpu.make_async_copy(g_hbm.at[src_v.at[0]], buf.at[b],
                                  gsem).wait()
            pltpu.async_copy(buf.at[b], acc_sh.at[dst_v.at[j]], ssem,
                             add=True)
            pltpu.async_copy(ones_v, deg_sh.at[dst_v.at[j]], dsem,
                             add=True)
            pltpu.make_async_copy(buf.at[b], acc_sh.at[dst_v.at[0]],
                                  ssem).wait()

            @pl.when(g + 1 < _CH1 // _NBUF)
            def _():
                pltpu.async_copy(g_hbm.at[src_v.at[j + _NBUF]], buf.at[b],
                                 gsem)
        return 0

    lax.fori_loop(0, _CH1 // _NBUF, group, 0)

    def ddrain(j, _):
        pltpu.make_async_copy(ones_v, deg_sh.at[dst_v.at[0]], dsem).wait()
        return 0

    lax.fori_loop(0, _CH1, ddrain, 0)

    plsc.subcore_barrier()

    for t in range(_RPT // _K):
        pltpu.sync_copy(acc_sh.at[pl.ds(base + t * _K, _K)],
                        acc_out.at[cid, pl.ds(base + t * _K, _K)])
        pltpu.sync_copy(deg_sh.at[pl.ds(base + t * _K, _K)],
                        deg_out.at[cid, pl.ds(base + t * _K, _K)])


def _prep_edges(ei, nsplit, chunks):
    tot = nsplit * chunks * _K
    fill = jnp.full((tot - _E,), _N, jnp.int32)
    src = jnp.concatenate([ei[0], fill]).reshape(nsplit, chunks, _K)
    dst = jnp.concatenate([ei[1], fill]).reshape(nsplit, chunks, _K)
    return src, dst


def _tc_fc(x, w, b):
    def body(x_ref, w_ref, b_ref, o_ref):
        r = jnp.maximum(
            jnp.dot(x_ref[...], w_ref[...],
                    preferred_element_type=jnp.float32) + b_ref[...], 0.0)
        o_ref[...] = jnp.stack([r[:, :64], r[:, 64:]], axis=0)

    return pl.pallas_call(
        body,
        grid=(10,),
        in_specs=[pl.BlockSpec((1024, _D), lambda i: (i, 0)),
                  pl.BlockSpec((_D, _H), lambda i: (0, 0)),
                  pl.BlockSpec((1, _H), lambda i: (0, 0))],
        out_specs=pl.BlockSpec((_NC, 1024, 64), lambda i: (0, i, 0)),
        out_shape=jax.ShapeDtypeStruct((_NC, _NPAD, 64), jnp.float32),
    )(x, w, b)


def _tc_mid(hs, acc, deg, ws0, wn0, b0, ws1, wn1):
    def body(h_ref, a_ref, d_ref, ws0_r, wn0_r, b0_r, ws1_r, wn1_r,
             s_ref, g_ref):
        h = jnp.concatenate([h_ref[0], h_ref[1]], axis=-1)
        d = jnp.maximum(d_ref[0, :, 0:1] + d_ref[1, :, 0:1], 1.0)
        mean = jnp.concatenate([a_ref[0], a_ref[1]], axis=-1) / d
        h1 = jnp.maximum(
            jnp.dot(h, ws0_r[...], preferred_element_type=jnp.float32)
            + jnp.dot(mean, wn0_r[...], preferred_element_type=jnp.float32)
            + b0_r[...], 0.0)
        s_ref[...] = jnp.dot(h1, ws1_r[...], preferred_element_type=jnp.float32)
        g_ref[...] = jnp.dot(h1, wn1_r[...], preferred_element_type=jnp.float32)

    return pl.pallas_call(
        body,
        grid=(10,),
        in_specs=[pl.BlockSpec((_NC, 1024, 64), lambda i: (0, i, 0)),
                  pl.BlockSpec((_NC, 1024, 64), lambda i: (0, i, 0)),
                  pl.BlockSpec((_NC, 1024, 16), lambda i: (0, i, 0)),
                  pl.BlockSpec((_H, _H), lambda i: (0, 0)),
                  pl.BlockSpec((_H, _H), lambda i: (0, 0)),
                  pl.BlockSpec((1, _H), lambda i: (0, 0)),
                  pl.BlockSpec((_H, _C), lambda i: (0, 0)),
                  pl.BlockSpec((_H, _C), lambda i: (0, 0))],
        out_specs=[pl.BlockSpec((1024, _C), lambda i: (i, 0)),
                   pl.BlockSpec((1024, _C), lambda i: (i, 0))],
        out_shape=[jax.ShapeDtypeStruct((_NPAD, _C), jnp.float32),
                   jax.ShapeDtypeStruct((_NPAD, _C), jnp.float32)],
    )(hs, acc, deg, ws0, wn0, b0, ws1, wn1)


def _tc_out(s2d, a2d, d2d, b1t):
    R = _NPAD // 8

    def body(s_ref, a_ref, d_ref, b_ref, o_ref):
        agg = ((a_ref[0] + a_ref[1])
               / jnp.maximum(d_ref[0] + d_ref[1], 1.0))
        o_ref[...] = s_ref[...] + agg + b_ref[...]

    return pl.pallas_call(
        body,
        grid=(1,),
        in_specs=[pl.BlockSpec((R, 128), lambda i: (0, 0)),
                  pl.BlockSpec((_NC, R, 128), lambda i: (0, 0, 0)),
                  pl.BlockSpec((_NC, R, 128), lambda i: (0, 0, 0)),
                  pl.BlockSpec((1, 128), lambda i: (0, 0))],
        out_specs=pl.BlockSpec((R, 128), lambda i: (0, 0)),
        out_shape=jax.ShapeDtypeStruct((R, 128), jnp.float32),
    )(s2d, a2d, d2d, b1t)


def kernel(x, edge_index0, edge_index1, W_fc, b_fc,
           W_self0, W_neigh0, b0, W_self1, W_neigh1, b1):
    xp = jnp.zeros((_NPAD, _D), jnp.float32).at[:_N].set(x)
    src0, dst0 = _prep_edges(edge_index0, _NS, _CH0)
    src1, dst1 = _prep_edges(edge_index1, _NW, _CH1)

    hs = _tc_fc(xp, W_fc, b_fc.reshape(1, _H))
    acc0, deg0 = _agg0(hs, src0, dst0)
    s, g = _tc_mid(hs, acc0, deg0, W_self0, W_neigh0, b0.reshape(1, _H),
                   W_self1, W_neigh1)
    acc1, deg1 = _agg1(g, src1, dst1)
    out2d = _tc_out(s.reshape(_NPAD // 8, 128),
                    acc1.reshape(_NC, _NPAD // 8, 128),
                    deg1.reshape(_NC, _NPAD // 8, 128),
                    jnp.tile(b1, 8).reshape(1, 128))
    return out2d.reshape(_NPAD, _C)[:_N]
